# baseline (speedup 1.0000x reference)
import jax
import jax.numpy as jnp
from jax import lax
from jax.experimental import pallas as pl
from jax.experimental.pallas import tpu as pltpu
from jax.experimental.pallas import tpu_sc as plsc

_IPW = 8
_L = 16


def _sc_body(user_hbm, items_hbm, q_hbm, emb_hbm, out_hbm,
             user_v, idx_v, uidx_v, q_v, qf_v, emb_v, emb_f, ret_v,
             sem_q, sem_e):
    n_workers = 200 // _IPW
    skills = q_hbm.shape[1]
    k_hidden = 64
    nkc = k_hidden // _L
    nsc = skills // _L
    wid = lax.axis_index("s") * 2 + lax.axis_index("c")
    iota = jnp.arange(_L, dtype=jnp.int32)

    @pl.when(wid < n_workers)
    def _():
        base = wid * _IPW
        pltpu.sync_copy(user_hbm, user_v)
        pltpu.sync_copy(items_hbm.at[pl.ds(base, _IPW)], idx_v)
        uv = user_v[...] * k_hidden
        for c in range(k_hidden // _L):
            uidx_v[pl.ds(c * _L, _L)] = uv + (iota + c * _L)
        emb_copy = pltpu.make_async_copy(emb_hbm.at[uidx_v], emb_v, sem_e)
        emb_copy.start()
        q_copy = pltpu.make_async_copy(q_hbm.at[idx_v], q_v, sem_q)
        q_copy.start()
        q_copy.wait()
        for i in range(_IPW):
            for c in range(nsc):
                qf_v[pl.ds(i * skills + c * _L, _L)] = (
                    q_v[i, pl.ds(c * _L, _L)].astype(jnp.float32))
        emb_copy.wait()
        for r in range(k_hidden):
            for c in range(nsc):
                plsc.store_scatter(
                    emb_f, [(iota + c * _L) * k_hidden + r],
                    emb_v[r, pl.ds(c * _L, _L)])

        unroll = 4
        for ip in range(_IPW // 2):
            i0, i1 = 2 * ip, 2 * ip + 1
            b0 = jnp.full((_L,), i0 * skills, jnp.int32)
            b1 = jnp.full((_L,), i1 * skills, jnp.int32)

            def sbody(s4, accs, b0=b0, b1=b1):
                soff = s4 * (unroll * k_hidden)
                sv = jnp.full((_L,), s4 * unroll, jnp.int32)
                new = list(accs)
                for u in range(unroll):
                    qb0 = plsc.load_gather(qf_v, [b0 + (sv + u)])
                    qb1 = plsc.load_gather(qf_v, [b1 + (sv + u)])
                    for c in range(nkc):
                        ev = emb_f[pl.ds(soff + u * k_hidden + c * _L, _L)]
                        new[c] = new[c] + qb0 * ev
                        new[nkc + c] = new[nkc + c] + qb1 * ev
                return tuple(new)

            accs0 = tuple(jnp.zeros((_L,), jnp.float32)
                          for _ in range(2 * nkc))
            accs = lax.fori_loop(0, skills // unroll, sbody, accs0)
            for c in range(nkc):
                ret_v[i0, pl.ds(c * _L, _L)] = accs[c]
                ret_v[i1, pl.ds(c * _L, _L)] = accs[nkc + c]
        pltpu.sync_copy(ret_v, out_hbm.at[pl.ds(base, _IPW)])


def kernel(user, Q_matrix, items, skill_embedding):
    seq_len = items.shape[0]
    n_items, skill_num = Q_matrix.shape
    k_hidden = skill_embedding.shape[2]
    n_users = skill_embedding.shape[0]
    user_arr = jnp.full((_L,), user, jnp.int32)
    emb_t = jnp.swapaxes(skill_embedding, 1, 2).reshape(
        n_users * k_hidden, skill_num)

    mesh = plsc.VectorSubcoreMesh(core_axis_name="c", subcore_axis_name="s")
    run = pl.kernel(
        _sc_body,
        out_type=jax.ShapeDtypeStruct((seq_len, k_hidden), jnp.float32),
        mesh=mesh,
        compiler_params=pltpu.CompilerParams(needs_layout_passes=False),
        scratch_types=[
            pltpu.VMEM((_L,), jnp.int32),
            pltpu.VMEM((_IPW,), jnp.int32),
            pltpu.VMEM((k_hidden,), jnp.int32),
            pltpu.VMEM((_IPW, skill_num), jnp.int32),
            pltpu.VMEM((_IPW * skill_num,), jnp.float32),
            pltpu.VMEM((k_hidden, skill_num), jnp.float32),
            pltpu.VMEM((k_hidden * skill_num,), jnp.float32),
            pltpu.VMEM((_IPW, k_hidden), jnp.float32),
            pltpu.SemaphoreType.DMA,
            pltpu.SemaphoreType.DMA,
        ],
    )
    return run(user_arr, items.astype(jnp.int32), Q_matrix, emb_t)


# speedup vs baseline: 10.0495x; 10.0495x over previous
import jax
import jax.numpy as jnp
from jax.experimental import pallas as pl
from jax.experimental.pallas import tpu as pltpu


def _body(user_ref, items_ref, q_ref, emb_hbm, out_ref, emb_vmem, sem):
    pltpu.make_async_copy(
        emb_hbm.at[pl.ds(user_ref[0], 1)], emb_vmem, sem).start()
    items_v = items_ref[...]
    seq_len = items_v.shape[0]
    n_items = q_ref.shape[0]
    col = jax.lax.broadcasted_iota(jnp.int32, (seq_len, n_items), 1)
    onehot = (col == items_v[:, None]).astype(jnp.float32)
    qf = q_ref[...].astype(jnp.float32)
    q_rows = jnp.dot(onehot, qf, preferred_element_type=jnp.float32)
    pltpu.make_async_copy(
        emb_hbm.at[pl.ds(user_ref[0], 1)], emb_vmem, sem).wait()
    emb_kt = emb_vmem[0]
    out_ref[...] = jax.lax.dot_general(
        emb_kt, q_rows, (((1,), (1,)), ((), ())),
        preferred_element_type=jnp.float32)


def kernel(user, Q_matrix, items, skill_embedding):
    seq_len = items.shape[0]
    n_items, skill_num = Q_matrix.shape
    k_hidden = skill_embedding.shape[2]
    user_arr = jnp.asarray(user, jnp.int32).reshape(1)
    emb_t = jnp.swapaxes(skill_embedding, 1, 2)

    grid_spec = pltpu.PrefetchScalarGridSpec(
        num_scalar_prefetch=1,
        grid=(1,),
        in_specs=[
            pl.BlockSpec((seq_len,), lambda i, u: (0,)),
            pl.BlockSpec((n_items, skill_num), lambda i, u: (0, 0)),
            pl.BlockSpec(memory_space=pl.ANY),
        ],
        out_specs=pl.BlockSpec((k_hidden, seq_len), lambda i, u: (0, 0)),
        scratch_shapes=[
            pltpu.VMEM((1, k_hidden, skill_num), jnp.float32),
            pltpu.SemaphoreType.DMA,
        ],
    )
    out_t = pl.pallas_call(
        _body,
        grid_spec=grid_spec,
        out_shape=jax.ShapeDtypeStruct((k_hidden, seq_len), jnp.float32),
    )(user_arr, items.astype(jnp.int32), Q_matrix, emb_t)
    return out_t.T
